# baseline (speedup 1.0000x reference)
import functools

import jax
import jax.numpy as jnp
from jax import lax
from jax.experimental import pallas as pl
from jax.experimental.pallas import tpu as pltpu
from jax.experimental.pallas import tpu_sc as plsc

_DIM = 4096
_NE = 64
_TOPK = 8
_T = 16384
_BT = 512
_NCHUNK = 2

_NC = 2
_NS = 16
_NW = _NC * _NS


def _matmul_body(x_ref, w_ref, o_ref):
    o_ref[...] = lax.dot_general(
        x_ref[...], w_ref[...],
        dimension_numbers=(((1,), (1,)), ((), ())),
        preferred_element_type=jnp.float32,
    )


def _scores_tc(x, weight, rows, row0):
    blk0 = row0 // _BT
    return pl.pallas_call(
        _matmul_body,
        grid=(rows // _BT,),
        in_specs=[
            pl.BlockSpec((_BT, _DIM), lambda i: (blk0 + i, 0)),
            pl.BlockSpec((_NE, _DIM), lambda i: (0, 0)),
        ],
        out_specs=pl.BlockSpec((_BT, _NE), lambda i: (i, 0)),
        out_shape=jax.ShapeDtypeStruct((rows, _NE), jnp.float32),
    )(x, weight)


@functools.cache
def _topk_sc(rows):
    nrow = rows // _NW

    def _topk_body(scores_hbm, wout_hbm, iout_hbm, sbuf, wbuf, ibuf):
        wid = lax.axis_index("s") * _NC + lax.axis_index("c")
        base = wid * nrow
        pltpu.sync_copy(scores_hbm.at[pl.ds(base, nrow)], sbuf)

        lanes = lax.iota(jnp.int32, 16)
        in_lo = lanes < 8

        def _merge(ka, va, kb, vb):
            kb_r = lax.rev(kb, (0,))
            vb_r = lax.rev(vb, (0,))
            k = jnp.where(in_lo, ka, kb_r)
            v = jnp.where(in_lo, va, vb_r)
            return plsc.sort_key_val(k, v, descending=True)

        @plsc.parallel_loop(0, nrow, step=1, unroll=8)
        def _row(r):
            srt = []
            for j in range(4):
                k = sbuf[r, pl.ds(16 * j, 16)]
                srt.append(
                    plsc.sort_key_val(k, lanes + 16 * j, descending=True))
            k01, v01 = _merge(*srt[0], *srt[1])
            k23, v23 = _merge(*srt[2], *srt[3])
            kf, vf = _merge(k01, v01, k23, v23)
            m = jnp.max(kf)
            e = jnp.exp(kf - m)
            e = jnp.where(in_lo, e, 0.0)
            s = jnp.broadcast_to(jnp.sum(e), (16,))
            wbuf[r] = e / s
            ibuf[r] = vf

        pltpu.sync_copy(wbuf, wout_hbm.at[pl.ds(base, nrow)])
        pltpu.sync_copy(ibuf, iout_hbm.at[pl.ds(base, nrow)])

    return pl.kernel(
        _topk_body,
        out_type=(
            jax.ShapeDtypeStruct((rows, 16), jnp.float32),
            jax.ShapeDtypeStruct((rows, 16), jnp.int32),
        ),
        mesh=plsc.VectorSubcoreMesh(core_axis_name="c", subcore_axis_name="s"),
        compiler_params=pltpu.CompilerParams(
            needs_layout_passes=False, use_tc_tiling_on_sc=False),
        scratch_types=[
            pltpu.VMEM((nrow, _NE), jnp.float32),
            pltpu.VMEM((nrow, 16), jnp.float32),
            pltpu.VMEM((nrow, 16), jnp.int32),
        ],
    )


def kernel(x, weight):
    crows = _T // _NCHUNK
    sc = _topk_sc(crows)
    outs = [sc(_scores_tc(x, weight, crows, c * crows))
            for c in range(_NCHUNK)]
    w16 = jnp.concatenate([o[0] for o in outs], axis=0)
    i16 = jnp.concatenate([o[1] for o in outs], axis=0)
    return (w16[:, :_TOPK], i16[:, :_TOPK])


# speedup vs baseline: 1.0086x; 1.0086x over previous
import jax
import jax.numpy as jnp
from jax import lax
from jax.experimental import pallas as pl
from jax.experimental.pallas import tpu as pltpu
from jax.experimental.pallas import tpu_sc as plsc

_DIM = 4096
_NE = 64
_TOPK = 8
_T = 16384
_BT = 512
_HB = _BT // 2

_NC = 2
_NS = 16
_NW = _NC * _NS
_PR = _T // 2 // _NW


def _matmul_body(x_ref, w_ref, o_ref):
    s = lax.dot_general(
        x_ref[...], w_ref[...],
        dimension_numbers=(((1,), (1,)), ((), ())),
        preferred_element_type=jnp.float32,
    )
    o_ref[...] = jnp.concatenate([s[:_HB], s[_HB:]], axis=1)


def _scores_tc(x, weight):
    return pl.pallas_call(
        _matmul_body,
        grid=(_T // _BT,),
        in_specs=[
            pl.BlockSpec((_BT, _DIM), lambda i: (i, 0)),
            pl.BlockSpec((_NE, _DIM), lambda i: (0, 0)),
        ],
        out_specs=pl.BlockSpec((_HB, 2 * _NE), lambda i: (i, 0)),
        out_shape=jax.ShapeDtypeStruct((_T // 2, 2 * _NE), jnp.float32),
    )(x, weight)


def _topk_body(scores_hbm, wout_hbm, iout_hbm, sbuf, wbuf, ibuf):
    wid = lax.axis_index("s") * _NC + lax.axis_index("c")
    base = wid * _PR
    pltpu.sync_copy(scores_hbm.at[pl.ds(base, _PR)], sbuf)

    lanes = lax.iota(jnp.int32, 16)
    in_lo = lanes < 8
    shift8 = (lanes + 8) & 15

    def _merge(ka, va, kb, vb):
        kb_r = lax.rev(kb, (0,))
        vb_r = lax.rev(vb, (0,))
        k = jnp.where(in_lo, ka, kb_r)
        v = jnp.where(in_lo, va, vb_r)
        return plsc.sort_key_val(k, v, descending=True)

    def _gather16(x, idx):
        return lax.gather(
            x, idx[:, None],
            lax.GatherDimensionNumbers(
                offset_dims=(), collapsed_slice_dims=(0,),
                start_index_map=(0,)),
            (1,), mode=lax.GatherScatterMode.PROMISE_IN_BOUNDS)

    def _top8(p, lane0):
        srt = []
        for j in range(4):
            k = sbuf[p, pl.ds(lane0 + 16 * j, 16)]
            srt.append(plsc.sort_key_val(k, lanes + 16 * j, descending=True))
        k01, v01 = _merge(*srt[0], *srt[1])
        k23, v23 = _merge(*srt[2], *srt[3])
        kf, vf = _merge(k01, v01, k23, v23)
        m = jnp.max(kf)
        e = jnp.exp(kf - m)
        e = jnp.where(in_lo, e, 0.0)
        s = jnp.broadcast_to(jnp.sum(e), (16,))
        return e / s, vf

    def _pair(w0, i0, w1, i1):
        w = jnp.where(in_lo, w0, _gather16(w1, shift8))
        i = jnp.where(in_lo, i0, _gather16(i1, shift8))
        return w, i

    @plsc.parallel_loop(0, _PR, step=2, unroll=4)
    def _rows(p):
        wa0, ia0 = _top8(p, 0)
        wa1, ia1 = _top8(p + 1, 0)
        wb0, ib0 = _top8(p, 64)
        wb1, ib1 = _top8(p + 1, 64)
        q = lax.div(p, 2)
        wbuf[q], ibuf[q] = _pair(wa0, ia0, wa1, ia1)
        half = _PR // 2
        wbuf[q + half], ibuf[q + half] = _pair(wb0, ib0, wb1, ib1)

    pltpu.sync_copy(wbuf, wout_hbm.at[pl.ds(base, _PR)])
    pltpu.sync_copy(ibuf, iout_hbm.at[pl.ds(base, _PR)])


_topk_sc = pl.kernel(
    _topk_body,
    out_type=(
        jax.ShapeDtypeStruct((_T // 2, 16), jnp.float32),
        jax.ShapeDtypeStruct((_T // 2, 16), jnp.int32),
    ),
    mesh=plsc.VectorSubcoreMesh(core_axis_name="c", subcore_axis_name="s"),
    compiler_params=pltpu.CompilerParams(
        needs_layout_passes=False, use_tc_tiling_on_sc=False),
    scratch_types=[
        pltpu.VMEM((_PR, 2 * _NE), jnp.float32),
        pltpu.VMEM((_PR, 16), jnp.float32),
        pltpu.VMEM((_PR, 16), jnp.int32),
    ],
)


def kernel(x, weight):
    scores = _scores_tc(x, weight)
    w2, i2 = _topk_sc(scores)
    return (w2.reshape(_T, _TOPK), i2.reshape(_T, _TOPK))


# speedup vs baseline: 1.0113x; 1.0027x over previous
import jax
import jax.numpy as jnp
from jax import lax
from jax.experimental import pallas as pl
from jax.experimental.pallas import tpu as pltpu
from jax.experimental.pallas import tpu_sc as plsc

_DIM = 4096
_NE = 64
_TOPK = 8
_T = 16384
_BT = 512
_HB = _BT // 2

_NC = 2
_NS = 16
_NW = _NC * _NS
_PR = _T // 2 // _NW


def _matmul_body(x_ref, w_ref, o_ref):
    s = lax.dot_general(
        x_ref[...], w_ref[...],
        dimension_numbers=(((1,), (1,)), ((), ())),
        preferred_element_type=jnp.float32,
    )
    o_ref[...] = jnp.concatenate([s[:_HB], s[_HB:]], axis=1)


def _scores_tc(x, weight):
    return pl.pallas_call(
        _matmul_body,
        grid=(_T // _BT,),
        in_specs=[
            pl.BlockSpec((_BT, _DIM), lambda i: (i, 0)),
            pl.BlockSpec((_NE, _DIM), lambda i: (0, 0)),
        ],
        out_specs=pl.BlockSpec((_HB, 2 * _NE), lambda i: (i, 0)),
        out_shape=jax.ShapeDtypeStruct((_T // 2, 2 * _NE), jnp.float32),
    )(x, weight)


def _topk_body(scores_hbm, wout_hbm, iout_hbm, sbuf, wbuf, ibuf):
    wid = lax.axis_index("s") * _NC + lax.axis_index("c")
    base = wid * _PR
    pltpu.sync_copy(scores_hbm.at[pl.ds(base, _PR)], sbuf)

    lanes = lax.iota(jnp.int32, 16)
    in_lo = lanes < 8

    def _merge(ka, va, kb, vb):
        kb_r = lax.rev(kb, (0,))
        vb_r = lax.rev(vb, (0,))
        k = jnp.where(in_lo, ka, kb_r)
        v = jnp.where(in_lo, va, vb_r)
        return plsc.sort_key_val(k, v, descending=True)

    def _top8(p, lane0):
        srt = []
        for j in range(4):
            k = sbuf[p, pl.ds(lane0 + 16 * j, 16)]
            srt.append(plsc.sort_key_val(k, lanes + 16 * j, descending=True))
        k01, v01 = _merge(*srt[0], *srt[1])
        k23, v23 = _merge(*srt[2], *srt[3])
        kf, vf = _merge(k01, v01, k23, v23)
        m = jnp.max(kf)
        e = jnp.exp(kf - m)
        e = jnp.where(in_lo, e, 0.0)
        s = jnp.broadcast_to(jnp.sum(e), (16,))
        return e / s, vf

    @plsc.parallel_loop(0, _PR, step=1, unroll=8)
    def _rows(p):
        for lane0, rt in ((0, p), (64, _PR + p)):
            w, v = _top8(p, lane0)
            rowv = jnp.broadcast_to(rt, (16,))
            plsc.store_scatter(wbuf, [rowv, lanes], w, mask=in_lo)
            plsc.store_scatter(ibuf, [rowv, lanes], v, mask=in_lo)

    pltpu.sync_copy(wbuf, wout_hbm.at[pl.ds(2 * base, 2 * _PR)])
    pltpu.sync_copy(ibuf, iout_hbm.at[pl.ds(2 * base, 2 * _PR)])


_topk_sc = pl.kernel(
    _topk_body,
    out_type=(
        jax.ShapeDtypeStruct((_T, _TOPK), jnp.float32),
        jax.ShapeDtypeStruct((_T, _TOPK), jnp.int32),
    ),
    mesh=plsc.VectorSubcoreMesh(core_axis_name="c", subcore_axis_name="s"),
    compiler_params=pltpu.CompilerParams(
        needs_layout_passes=False, use_tc_tiling_on_sc=False),
    scratch_types=[
        pltpu.VMEM((_PR, 2 * _NE), jnp.float32),
        pltpu.VMEM((2 * _PR, _TOPK), jnp.float32),
        pltpu.VMEM((2 * _PR, _TOPK), jnp.int32),
    ],
)


def kernel(x, weight):
    scores = _scores_tc(x, weight)
    return _topk_sc(scores)


# speedup vs baseline: 1.0238x; 1.0124x over previous
import jax
import jax.numpy as jnp
from jax import lax
from jax.experimental import pallas as pl
from jax.experimental.pallas import tpu as pltpu
from jax.experimental.pallas import tpu_sc as plsc

_DIM = 4096
_NE = 64
_TOPK = 8
_T = 16384
_BT = 512
_HB = _BT // 2

_NC = 2
_NS = 16
_NW = _NC * _NS
_PR = _T // 2 // _NW


def _matmul_body(x_ref, w_ref, o_ref):
    s = lax.dot_general(
        x_ref[...], w_ref[...],
        dimension_numbers=(((1,), (1,)), ((), ())),
        preferred_element_type=jnp.float32,
    )
    o_ref[...] = jnp.concatenate([s[:_HB], s[_HB:]], axis=1)


def _scores_tc(x, weight):
    return pl.pallas_call(
        _matmul_body,
        grid=(_T // _BT,),
        in_specs=[
            pl.BlockSpec((_BT, _DIM), lambda i: (i, 0)),
            pl.BlockSpec((_NE, _DIM), lambda i: (0, 0)),
        ],
        out_specs=pl.BlockSpec((_HB, 2 * _NE), lambda i: (i, 0)),
        out_shape=jax.ShapeDtypeStruct((_T // 2, 2 * _NE), jnp.float32),
    )(x, weight)


_NPASS = 4
_PP = _PR // _NPASS


def _topk_body(scores_hbm, wout_hbm, iout_hbm, sbuf, wbuf, ibuf):
    wid = lax.axis_index("s") * _NC + lax.axis_index("c")
    base = wid * _PR

    lanes = lax.iota(jnp.int32, 16)
    in_lo = lanes < 8

    def _merge(ka, va, kb, vb):
        kb_r = lax.rev(kb, (0,))
        vb_r = lax.rev(vb, (0,))
        k = jnp.where(in_lo, ka, kb_r)
        v = jnp.where(in_lo, va, vb_r)
        return plsc.sort_key_val(k, v, descending=True)

    def _top8(p, lane0):
        srt = []
        for j in range(4):
            k = sbuf[p, pl.ds(lane0 + 16 * j, 16)]
            srt.append(plsc.sort_key_val(k, lanes + 16 * j, descending=True))
        k01, v01 = _merge(*srt[0], *srt[1])
        k23, v23 = _merge(*srt[2], *srt[3])
        kf, vf = _merge(k01, v01, k23, v23)
        m = jnp.max(kf)
        e = jnp.exp(kf - m)
        e = jnp.where(in_lo, e, 0.0)
        s = jnp.broadcast_to(jnp.sum(e), (16,))
        return e / s, vf

    for ps in range(_NPASS):
        pltpu.sync_copy(scores_hbm.at[pl.ds(base + ps * _PP, _PP)], sbuf)

        @plsc.parallel_loop(0, _PP, step=1, unroll=8)
        def _rows(p):
            for lane0, rt in ((0, p), (64, _PP + p)):
                w, v = _top8(p, lane0)
                rowv = jnp.broadcast_to(rt, (16,))
                plsc.store_scatter(wbuf, [rowv, lanes], w, mask=in_lo)
                plsc.store_scatter(ibuf, [rowv, lanes], v, mask=in_lo)

        a0 = 2 * base + ps * _PP
        b0 = 2 * base + _PR + ps * _PP
        pltpu.sync_copy(wbuf.at[pl.ds(0, _PP)], wout_hbm.at[pl.ds(a0, _PP)])
        pltpu.sync_copy(wbuf.at[pl.ds(_PP, _PP)], wout_hbm.at[pl.ds(b0, _PP)])
        pltpu.sync_copy(ibuf.at[pl.ds(0, _PP)], iout_hbm.at[pl.ds(a0, _PP)])
        pltpu.sync_copy(ibuf.at[pl.ds(_PP, _PP)], iout_hbm.at[pl.ds(b0, _PP)])


_topk_sc = pl.kernel(
    _topk_body,
    out_type=(
        jax.ShapeDtypeStruct((_T, _TOPK), jnp.float32),
        jax.ShapeDtypeStruct((_T, _TOPK), jnp.int32),
    ),
    mesh=plsc.VectorSubcoreMesh(core_axis_name="c", subcore_axis_name="s"),
    compiler_params=pltpu.CompilerParams(
        needs_layout_passes=False, use_tc_tiling_on_sc=True),
    scratch_types=[
        pltpu.VMEM((_PP, 2 * _NE), jnp.float32),
        pltpu.VMEM((2 * _PP, _TOPK), jnp.float32),
        pltpu.VMEM((2 * _PP, _TOPK), jnp.int32),
    ],
)


def kernel(x, weight):
    scores = _scores_tc(x, weight)
    return _topk_sc(scores)
